# Initial kernel scaffold; baseline (speedup 1.0000x reference)
import jax
import jax.numpy as jnp
from jax.experimental import pallas as pl


def kernel(q, kv, proj_dir, sampled_set):
    raise NotImplementedError("write your pallas kernel here")


# speedup vs baseline: 7.0987x; 7.0987x over previous
import functools
import math

import jax
import jax.numpy as jnp
import numpy as np
from jax.experimental import pallas as pl

HEAD_DIM = 64
NUM_PROJS = 8
BLK = 128
SAMPLE = 128
F32_MIN = float(np.finfo(np.float32).min)


def _attn_body(q_ref, k_ref, v_ref, ksub_ref, vsub_ref, sset_ref, o_ref):
    scale = HEAD_DIM ** -0.5
    q = q_ref[0, 0]
    k = k_ref[0, 0]
    v = v_ref[0, 0]
    dot_t = lambda a, b: jax.lax.dot_general(
        a, b, (((1,), (1,)), ((), ())), preferred_element_type=jnp.float32)
    qk = dot_t(q, k) * scale
    m1 = jnp.max(qk, axis=-1, keepdims=True)
    e1 = jnp.exp(qk - m1)
    s1 = jnp.sum(e1, axis=-1, keepdims=True)
    a1 = jnp.dot(e1, v, preferred_element_type=jnp.float32) / s1
    lse1 = jnp.log(s1) + m1
    j = pl.program_id(1)
    qk2 = dot_t(q, ksub_ref[0]) * scale
    kblk = sset_ref[0, 0] // BLK
    qk2 = jnp.where((kblk == j)[None, :], F32_MIN, qk2)
    m2 = jnp.max(qk2, axis=-1, keepdims=True)
    e2 = jnp.exp(qk2 - m2)
    s2 = jnp.sum(e2, axis=-1, keepdims=True)
    a2 = jnp.dot(e2, vsub_ref[0], preferred_element_type=jnp.float32) / s2
    lse2 = jnp.log(s2) + m2 + math.log(8192.0 / SAMPLE)
    c = jax.nn.sigmoid(lse1 - lse2)
    o_ref[0, 0] = c * a1 + (1.0 - c) * a2


def _fused_attention(qs, ks, vs, ksub, vsub, sset, interpret=False):
    BH, NB, _, D = qs.shape
    grid = (BH, NB)
    blk4 = pl.BlockSpec((1, 1, BLK, D), lambda b, j: (b, j, 0, 0))
    sub3 = pl.BlockSpec((1, SAMPLE, D), lambda b, j: (b, 0, 0))
    idx3 = pl.BlockSpec((1, 1, SAMPLE), lambda b, j: (b, 0, 0))
    return pl.pallas_call(
        _attn_body,
        grid=grid,
        in_specs=[blk4, blk4, blk4, sub3, sub3, idx3],
        out_specs=blk4,
        out_shape=jax.ShapeDtypeStruct((BH, NB, BLK, D), jnp.float32),
        interpret=interpret,
    )(qs, ks, vs, ksub, vsub, sset)


def _hash(x, proj):
    bits = (jnp.einsum('bsd,dp->bsp', x, proj) > 0).astype(jnp.int32)
    enc = (2 ** jnp.arange(NUM_PROJS, dtype=jnp.int32))
    b = jnp.sum(bits * enc, axis=-1)
    return b ^ (b >> 1)


def kernel(q, kv, proj_dir, sampled_set, interpret=False):
    B, H, S, D = q.shape
    BH = B * H
    NB = S // BLK
    q3 = q.reshape(BH, S, D)
    k3 = kv[:, :, 0].reshape(BH, S, D)
    v3 = kv[:, :, 1].reshape(BH, S, D)
    proj = proj_dir[0, 0]
    qh = _hash(q3, proj)
    kh = _hash(k3, proj)
    q_idx = jnp.argsort(qh, axis=1)
    k_idx = jnp.argsort(kh, axis=1)
    take = lambda x, i: jnp.take_along_axis(x, i[..., None], axis=1)
    qs = take(q3, q_idx)
    ks = take(k3, k_idx)
    vs = take(v3, k_idx)
    sset = sampled_set.reshape(BH, SAMPLE)
    ksub = jnp.take_along_axis(ks, sset[..., None], axis=1)
    vsub = jnp.take_along_axis(vs, sset[..., None], axis=1)
    attn_sorted = _fused_attention(
        qs.reshape(BH, NB, BLK, D), ks.reshape(BH, NB, BLK, D),
        vs.reshape(BH, NB, BLK, D), ksub, vsub,
        sset.reshape(BH, 1, SAMPLE), interpret=interpret)
    attn_sorted = attn_sorted.reshape(BH, S, D)
    inv = jnp.argsort(q_idx, axis=1)
    out = take(attn_sorted, inv)
    return out.reshape(B, H, S, D)
